# baseline (speedup 1.0000x reference)
import functools

import jax
import jax.numpy as jnp
import numpy as np
from jax.experimental import pallas as pl
from jax.experimental.pallas import tpu as pltpu

H = 128
R = 32
CUT = 0.5
CLO = 0.0
MAXZ_PAD = 128
BE = 1024
WN = 64


def _edge_scatter_block(eb_r, nb_r, fi_r, va_r,
                        src_r, az_r, fr_r,
                        embP_r, embQ_r, bz_r, wias_r, bias_r,
                        out_r):
    p = pl.program_id(0)
    j = nb_r[p]
    first = fi_r[p]
    valid = va_r[p]

    be = src_r.shape[2]
    src = src_r[...].reshape(1, be)

    az = az_r[...].reshape(be, 2)
    azs = az[:, 0:1]
    azd = az[:, 1:2]
    ohS = (jax.lax.broadcasted_iota(jnp.int32, (be, MAXZ_PAD), 1) == azs
           ).astype(jnp.float32)
    ohD = (jax.lax.broadcasted_iota(jnp.int32, (be, MAXZ_PAD), 1) == azd
           ).astype(jnp.float32)
    zij = (jnp.dot(ohS, embP_r[...], preferred_element_type=jnp.float32)
           + jnp.dot(ohD, embQ_r[...], preferred_element_type=jnp.float32)
           + bz_r[...])

    fr = fr_r[...].reshape(be, 4)
    d = fr[:, 0:1]
    rcut = 0.5 * (jnp.cos(d * (np.pi / CUT)) + 1.0) * (d < CUT)
    alpha = 5.0 / (CUT - CLO)
    start = float(np.exp(-(CUT - CLO)))
    step = (1.0 - start) / (R - 1)
    means = (start + step *
             jax.lax.broadcasted_iota(jnp.int32, (1, R), 1).astype(jnp.float32))
    beta = (2.0 / R * (1.0 - start)) ** -2
    ex = jnp.exp(alpha * (CLO - d))
    rfv = jnp.exp(-beta * (ex - means) ** 2) * rcut

    pIAS = (jnp.dot(rfv, wias_r[...], preferred_element_type=jnp.float32)
            + bias_r[...])
    base = rcut * zij
    uI = pIAS[:, 0 * H:1 * H] * base
    uA = pIAS[:, 1 * H:2 * H] * base
    uS = pIAS[:, 2 * H:3 * H] * base

    dinv = 1.0 / d
    rxn = fr[:, 1:2] * dinv
    ryn = fr[:, 2:3] * dinv
    rzn = fr[:, 3:4] * dinv
    tr3 = (rxn * rxn + ryn * ryn + rzn * rzn) * (1.0 / 3.0)

    M = jnp.concatenate([
        uI,
        rxn * uA, ryn * uA, rzn * uA,
        (rxn * rxn - tr3) * uS, (ryn * ryn - tr3) * uS, (rzn * rzn - tr3) * uS,
        (rxn * ryn) * uS, (rxn * rzn) * uS, (ryn * rzn) * uS,
    ], axis=1)

    rows = jax.lax.broadcasted_iota(jnp.int32, (WN, be), 0) + j * WN
    sel = (rows == src).astype(jnp.float32) * (valid == 1).astype(jnp.float32)
    contrib = jnp.dot(sel, M, preferred_element_type=jnp.float32)

    @pl.when(first == 1)
    def _():
        out_r[...] = jnp.zeros_like(out_r)

    out_r[...] += contrib


def _edge_scatter(eb, nb, fi, va, src3, az3, fr3,
                  embP, embQ, bz, wias, bias, npad, interpret=False):
    n_pairs = eb.shape[0]
    be = src3.shape[2]
    full = lambda shp: pl.BlockSpec(shp, lambda p, e, n, f, v: (0, 0))
    grid_spec = pltpu.PrefetchScalarGridSpec(
        num_scalar_prefetch=4,
        grid=(n_pairs,),
        in_specs=[
            pl.BlockSpec((1, 1, be), lambda p, e, n, f, v: (e[p], 0, 0)),
            pl.BlockSpec((1, be, 2), lambda p, e, n, f, v: (e[p], 0, 0)),
            pl.BlockSpec((1, be, 4), lambda p, e, n, f, v: (e[p], 0, 0)),
            full((MAXZ_PAD, H)), full((MAXZ_PAD, H)), full((1, H)),
            full((R, 3 * H)), full((1, 3 * H)),
        ],
        out_specs=pl.BlockSpec((WN, 10 * H), lambda p, e, n, f, v: (n[p], 0)),
    )
    return pl.pallas_call(
        _edge_scatter_block,
        grid_spec=grid_spec,
        out_shape=jax.ShapeDtypeStruct((npad, 10 * H), jnp.float32),
        compiler_params=pltpu.CompilerParams(
            dimension_semantics=("arbitrary",)),
        interpret=interpret,
    )(eb, nb, fi, va, src3, az3, fr3, embP, embQ, bz, wias, bias)


def _node_phase_block(acc_ref, wt0_ref, wt1_ref, wt2_ref, ws1_ref, bs1_ref,
                      ws2_ref, bs2_ref, g_ref, b_ref, out_ref):
    acc = acc_ref[...]
    sI = acc[:, 0, :]
    w0, w1, w2 = acc[:, 1, :], acc[:, 2, :], acc[:, 3, :]
    mxx, myy, mzz = acc[:, 4, :], acc[:, 5, :], acc[:, 6, :]
    mxy, mxz, myz = acc[:, 7, :], acc[:, 8, :], acc[:, 9, :]

    frob = (3.0 * sI * sI + 2.0 * (w0 * w0 + w1 * w1 + w2 * w2)
            + (mxx * mxx + myy * myy + mzz * mzz)
            + 2.0 * (mxy * mxy + mxz * mxz + myz * myz))

    mu = jnp.mean(frob, axis=-1, keepdims=True)
    var = jnp.mean((frob - mu) ** 2, axis=-1, keepdims=True)
    x = (frob - mu) * jax.lax.rsqrt(var + 1e-5) * g_ref[...] + b_ref[...]

    h1 = x @ ws1_ref[...] + bs1_ref[...]
    h1 = h1 * jax.nn.sigmoid(h1)
    h2 = h1 @ ws2_ref[...] + bs2_ref[...]
    h2 = h2 * jax.nn.sigmoid(h2)
    n0 = h2[:, 0 * H:1 * H]
    n1 = h2[:, 1 * H:2 * H]
    n2 = h2[:, 2 * H:3 * H]

    wt0 = wt0_ref[...]
    wt1 = wt1_ref[...]
    wt2 = wt2_ref[...]
    sIp = (sI @ wt0) * n0
    w0p = (w0 @ wt1) * n1
    w1p = (w1 @ wt1) * n1
    w2p = (w2 @ wt1) * n1
    mxxp = (mxx @ wt2) * n2
    myyp = (myy @ wt2) * n2
    mzzp = (mzz @ wt2) * n2
    mxyp = (mxy @ wt2) * n2
    mxzp = (mxz @ wt2) * n2
    myzp = (myz @ wt2) * n2

    out_ref[:, 0, :] = sIp + mxxp
    out_ref[:, 1, :] = -w2p + mxyp
    out_ref[:, 2, :] = w1p + mxzp
    out_ref[:, 3, :] = w2p + mxyp
    out_ref[:, 4, :] = sIp + myyp
    out_ref[:, 5, :] = -w0p + myzp
    out_ref[:, 6, :] = -w1p + mxzp
    out_ref[:, 7, :] = w0p + myzp
    out_ref[:, 8, :] = sIp + mzzp


def _node_phase(acc, W_t0, W_t1, W_t2, W_s1, b_s1, W_s2p, b_s2p, ln_g, ln_b,
                interpret=False):
    npad = acc.shape[0]
    bn = 64
    grid = (npad // bn,)
    full = lambda shp: pl.BlockSpec(shp, lambda i: (0,) * len(shp))
    return pl.pallas_call(
        _node_phase_block,
        grid=grid,
        in_specs=[
            pl.BlockSpec((bn, 10, H), lambda i: (i, 0, 0)),
            full((H, H)), full((H, H)), full((H, H)),
            full((H, 2 * H)), full((2 * H,)),
            full((2 * H, 3 * H)), full((3 * H,)),
            full((H,)), full((H,)),
        ],
        out_specs=pl.BlockSpec((bn, 9, H), lambda i: (i, 0, 0)),
        out_shape=jax.ShapeDtypeStruct((npad, 9, H), jnp.float32),
        interpret=interpret,
    )(acc, W_t0, W_t1, W_t2, W_s1, b_s1, W_s2p, b_s2p, ln_g, ln_b)


def kernel(atomic_numbers, pair_indices, d_ij, r_ij, emb, W_zij, b_zij,
           W_I, b_I, W_A, b_A, W_S, b_S, W_t0, W_t1, W_t2,
           W_s1, b_s1, W_s2, b_s2, ln_g, ln_b, *, interpret=False):
    n = atomic_numbers.shape[0]
    e = d_ij.shape[0]
    src = pair_indices[0]
    dst = pair_indices[1]

    order = jnp.argsort(src)
    srcs = src[order]
    azs = atomic_numbers[srcs].astype(jnp.int32)
    azd = atomic_numbers[dst[order]].astype(jnp.int32)
    ds = d_ij[:, 0][order]
    rs = r_ij[order]

    nEb = (e + BE - 1) // BE
    epad = nEb * BE
    npad = ((n + WN - 1) // WN) * WN
    nNb = npad // WN

    pe = epad - e
    srcp = jnp.pad(srcs.astype(jnp.int32), (0, pe), constant_values=n)
    azsp = jnp.pad(azs, (0, pe))
    azdp = jnp.pad(azd, (0, pe))
    dp = jnp.pad(ds, (0, pe), constant_values=1.0)
    rxp = jnp.pad(rs[:, 0], (0, pe))
    ryp = jnp.pad(rs[:, 1], (0, pe))
    rzp = jnp.pad(rs[:, 2], (0, pe))

    src3 = srcp.reshape(nEb, 1, BE)
    az3 = jnp.stack([azsp, azdp], axis=-1).reshape(nEb, BE, 2)
    fr3 = jnp.stack([dp, rxp, ryp, rzp], axis=-1).reshape(nEb, BE, 4)

    wb = jnp.arange(nNb, dtype=jnp.int32)
    e0 = jnp.searchsorted(srcp, wb * WN).astype(jnp.int32)
    e1 = jnp.searchsorted(srcp, wb * WN + WN).astype(jnp.int32)
    empty = e1 == e0
    b0 = jnp.clip(e0 // BE, 0, nEb - 1)
    b1 = jnp.where(empty, b0, jnp.clip((e1 - 1) // BE, 0, nEb - 1))
    cnt = jnp.where(empty, 1, b1 - b0 + 1)
    starts = jnp.concatenate([jnp.zeros(1, jnp.int32),
                              jnp.cumsum(cnt)[:-1].astype(jnp.int32)])

    n_pairs = nEb + nNb - 1
    p = jnp.arange(n_pairs, dtype=jnp.int32)
    j = jnp.clip(jnp.searchsorted(starts, p, side='right').astype(jnp.int32)
                 - 1, 0, nNb - 1)
    boff = p - starts[j]
    eb = jnp.clip(b0[j] + boff, 0, nEb - 1).astype(jnp.int32)
    va = ((~empty[j]) & (boff <= b1[j] - b0[j])).astype(jnp.int32)
    nb = j
    fi = (p == starts[j]).astype(jnp.int32)

    embP = (emb @ W_zij[:H]).astype(jnp.float32)
    embQ = (emb @ W_zij[H:]).astype(jnp.float32)
    embP = jnp.pad(embP, ((0, MAXZ_PAD - embP.shape[0]), (0, 0)))
    embQ = jnp.pad(embQ, ((0, MAXZ_PAD - embQ.shape[0]), (0, 0)))
    bz = b_zij.reshape(1, H)
    wias = jnp.concatenate([W_I, W_A, W_S], axis=1)
    bias = jnp.concatenate([b_I, b_A, b_S]).reshape(1, 3 * H)

    acc = _edge_scatter(eb, nb, fi, va, src3, az3, fr3,
                        embP, embQ, bz, wias, bias, npad,
                        interpret=interpret)
    acc = acc.reshape(npad, 10, H)

    perm = jnp.arange(3 * H).reshape(H, 3).T.reshape(-1)
    W_s2p = W_s2[:, perm]
    b_s2p = b_s2[perm]

    out9 = _node_phase(acc, W_t0, W_t1, W_t2, W_s1, b_s1, W_s2p, b_s2p,
                       ln_g, ln_b, interpret=interpret)
    out = jnp.transpose(out9[:n], (0, 2, 1)).reshape(n, H, 3, 3)
    return out


# speedup vs baseline: 1.0040x; 1.0040x over previous
import functools

import jax
import jax.numpy as jnp
import numpy as np
from jax.experimental import pallas as pl
from jax.experimental.pallas import tpu as pltpu

H = 128
R = 32
CUT = 0.5
CLO = 0.0
MAXZ_PAD = 128
BE = 2048
WN = 128


def _edge_scatter_block(eb_r, nb_r, fi_r, va_r,
                        src_r, az_r, fr_r,
                        embP_r, embQ_r, bz_r, wias_r, bias_r,
                        out_r):
    p = pl.program_id(0)
    j = nb_r[p]
    first = fi_r[p]
    valid = va_r[p]

    be = src_r.shape[2]
    src = src_r[...].reshape(1, be)

    az = az_r[...].reshape(be, 2)
    azs = az[:, 0:1]
    azd = az[:, 1:2]
    ohS = (jax.lax.broadcasted_iota(jnp.int32, (be, MAXZ_PAD), 1) == azs
           ).astype(jnp.float32)
    ohD = (jax.lax.broadcasted_iota(jnp.int32, (be, MAXZ_PAD), 1) == azd
           ).astype(jnp.float32)
    zij = (jnp.dot(ohS, embP_r[...], preferred_element_type=jnp.float32)
           + jnp.dot(ohD, embQ_r[...], preferred_element_type=jnp.float32)
           + bz_r[...])

    fr = fr_r[...].reshape(be, 4)
    d = fr[:, 0:1]
    rcut = 0.5 * (jnp.cos(d * (np.pi / CUT)) + 1.0) * (d < CUT)
    alpha = 5.0 / (CUT - CLO)
    start = float(np.exp(-(CUT - CLO)))
    step = (1.0 - start) / (R - 1)
    means = (start + step *
             jax.lax.broadcasted_iota(jnp.int32, (1, R), 1).astype(jnp.float32))
    beta = (2.0 / R * (1.0 - start)) ** -2
    ex = jnp.exp(alpha * (CLO - d))
    rfv = jnp.exp(-beta * (ex - means) ** 2) * rcut

    pIAS = (jnp.dot(rfv, wias_r[...], preferred_element_type=jnp.float32)
            + bias_r[...])
    base = rcut * zij
    uI = pIAS[:, 0 * H:1 * H] * base
    uA = pIAS[:, 1 * H:2 * H] * base
    uS = pIAS[:, 2 * H:3 * H] * base

    dinv = 1.0 / d
    rxn = fr[:, 1:2] * dinv
    ryn = fr[:, 2:3] * dinv
    rzn = fr[:, 3:4] * dinv
    tr3 = (rxn * rxn + ryn * ryn + rzn * rzn) * (1.0 / 3.0)

    M = jnp.concatenate([
        uI,
        rxn * uA, ryn * uA, rzn * uA,
        (rxn * rxn - tr3) * uS, (ryn * ryn - tr3) * uS, (rzn * rzn - tr3) * uS,
        (rxn * ryn) * uS, (rxn * rzn) * uS, (ryn * rzn) * uS,
    ], axis=1)

    rows = jax.lax.broadcasted_iota(jnp.int32, (WN, be), 0) + j * WN
    sel = (rows == src).astype(jnp.float32) * (valid == 1).astype(jnp.float32)
    contrib = jnp.dot(sel, M, preferred_element_type=jnp.float32)

    @pl.when(first == 1)
    def _():
        out_r[...] = jnp.zeros_like(out_r)

    out_r[...] += contrib


def _edge_scatter(eb, nb, fi, va, src3, az3, fr3,
                  embP, embQ, bz, wias, bias, npad, interpret=False):
    n_pairs = eb.shape[0]
    be = src3.shape[2]
    full = lambda shp: pl.BlockSpec(shp, lambda p, e, n, f, v: (0, 0))
    grid_spec = pltpu.PrefetchScalarGridSpec(
        num_scalar_prefetch=4,
        grid=(n_pairs,),
        in_specs=[
            pl.BlockSpec((1, 1, be), lambda p, e, n, f, v: (e[p], 0, 0)),
            pl.BlockSpec((1, be, 2), lambda p, e, n, f, v: (e[p], 0, 0)),
            pl.BlockSpec((1, be, 4), lambda p, e, n, f, v: (e[p], 0, 0)),
            full((MAXZ_PAD, H)), full((MAXZ_PAD, H)), full((1, H)),
            full((R, 3 * H)), full((1, 3 * H)),
        ],
        out_specs=pl.BlockSpec((WN, 10 * H), lambda p, e, n, f, v: (n[p], 0)),
    )
    return pl.pallas_call(
        _edge_scatter_block,
        grid_spec=grid_spec,
        out_shape=jax.ShapeDtypeStruct((npad, 10 * H), jnp.float32),
        compiler_params=pltpu.CompilerParams(
            dimension_semantics=("arbitrary",)),
        interpret=interpret,
    )(eb, nb, fi, va, src3, az3, fr3, embP, embQ, bz, wias, bias)


def _node_phase_block(acc_ref, wt0_ref, wt1_ref, wt2_ref, ws1_ref, bs1_ref,
                      ws2_ref, bs2_ref, g_ref, b_ref, out_ref):
    acc = acc_ref[...]
    sI = acc[:, 0, :]
    w0, w1, w2 = acc[:, 1, :], acc[:, 2, :], acc[:, 3, :]
    mxx, myy, mzz = acc[:, 4, :], acc[:, 5, :], acc[:, 6, :]
    mxy, mxz, myz = acc[:, 7, :], acc[:, 8, :], acc[:, 9, :]

    frob = (3.0 * sI * sI + 2.0 * (w0 * w0 + w1 * w1 + w2 * w2)
            + (mxx * mxx + myy * myy + mzz * mzz)
            + 2.0 * (mxy * mxy + mxz * mxz + myz * myz))

    mu = jnp.mean(frob, axis=-1, keepdims=True)
    var = jnp.mean((frob - mu) ** 2, axis=-1, keepdims=True)
    x = (frob - mu) * jax.lax.rsqrt(var + 1e-5) * g_ref[...] + b_ref[...]

    h1 = x @ ws1_ref[...] + bs1_ref[...]
    h1 = h1 * jax.nn.sigmoid(h1)
    h2 = h1 @ ws2_ref[...] + bs2_ref[...]
    h2 = h2 * jax.nn.sigmoid(h2)
    n0 = h2[:, 0 * H:1 * H]
    n1 = h2[:, 1 * H:2 * H]
    n2 = h2[:, 2 * H:3 * H]

    wt0 = wt0_ref[...]
    wt1 = wt1_ref[...]
    wt2 = wt2_ref[...]
    sIp = (sI @ wt0) * n0
    w0p = (w0 @ wt1) * n1
    w1p = (w1 @ wt1) * n1
    w2p = (w2 @ wt1) * n1
    mxxp = (mxx @ wt2) * n2
    myyp = (myy @ wt2) * n2
    mzzp = (mzz @ wt2) * n2
    mxyp = (mxy @ wt2) * n2
    mxzp = (mxz @ wt2) * n2
    myzp = (myz @ wt2) * n2

    out_ref[:, 0, :] = sIp + mxxp
    out_ref[:, 1, :] = -w2p + mxyp
    out_ref[:, 2, :] = w1p + mxzp
    out_ref[:, 3, :] = w2p + mxyp
    out_ref[:, 4, :] = sIp + myyp
    out_ref[:, 5, :] = -w0p + myzp
    out_ref[:, 6, :] = -w1p + mxzp
    out_ref[:, 7, :] = w0p + myzp
    out_ref[:, 8, :] = sIp + mzzp


def _node_phase(acc, W_t0, W_t1, W_t2, W_s1, b_s1, W_s2p, b_s2p, ln_g, ln_b,
                interpret=False):
    npad = acc.shape[0]
    bn = 64
    grid = (npad // bn,)
    full = lambda shp: pl.BlockSpec(shp, lambda i: (0,) * len(shp))
    return pl.pallas_call(
        _node_phase_block,
        grid=grid,
        in_specs=[
            pl.BlockSpec((bn, 10, H), lambda i: (i, 0, 0)),
            full((H, H)), full((H, H)), full((H, H)),
            full((H, 2 * H)), full((2 * H,)),
            full((2 * H, 3 * H)), full((3 * H,)),
            full((H,)), full((H,)),
        ],
        out_specs=pl.BlockSpec((bn, 9, H), lambda i: (i, 0, 0)),
        out_shape=jax.ShapeDtypeStruct((npad, 9, H), jnp.float32),
        interpret=interpret,
    )(acc, W_t0, W_t1, W_t2, W_s1, b_s1, W_s2p, b_s2p, ln_g, ln_b)


def kernel(atomic_numbers, pair_indices, d_ij, r_ij, emb, W_zij, b_zij,
           W_I, b_I, W_A, b_A, W_S, b_S, W_t0, W_t1, W_t2,
           W_s1, b_s1, W_s2, b_s2, ln_g, ln_b, *, interpret=False):
    n = atomic_numbers.shape[0]
    e = d_ij.shape[0]
    src = pair_indices[0]
    dst = pair_indices[1]

    order = jnp.argsort(src)
    srcs = src[order]
    azs = atomic_numbers[srcs].astype(jnp.int32)
    azd = atomic_numbers[dst[order]].astype(jnp.int32)
    ds = d_ij[:, 0][order]
    rs = r_ij[order]

    nEb = (e + BE - 1) // BE
    epad = nEb * BE
    npad = ((n + WN - 1) // WN) * WN
    nNb = npad // WN

    pe = epad - e
    srcp = jnp.pad(srcs.astype(jnp.int32), (0, pe), constant_values=n)
    azsp = jnp.pad(azs, (0, pe))
    azdp = jnp.pad(azd, (0, pe))
    dp = jnp.pad(ds, (0, pe), constant_values=1.0)
    rxp = jnp.pad(rs[:, 0], (0, pe))
    ryp = jnp.pad(rs[:, 1], (0, pe))
    rzp = jnp.pad(rs[:, 2], (0, pe))

    src3 = srcp.reshape(nEb, 1, BE)
    az3 = jnp.stack([azsp, azdp], axis=-1).reshape(nEb, BE, 2)
    fr3 = jnp.stack([dp, rxp, ryp, rzp], axis=-1).reshape(nEb, BE, 4)

    wb = jnp.arange(nNb, dtype=jnp.int32)
    e0 = jnp.searchsorted(srcp, wb * WN).astype(jnp.int32)
    e1 = jnp.searchsorted(srcp, wb * WN + WN).astype(jnp.int32)
    empty = e1 == e0
    b0 = jnp.clip(e0 // BE, 0, nEb - 1)
    b1 = jnp.where(empty, b0, jnp.clip((e1 - 1) // BE, 0, nEb - 1))
    cnt = jnp.where(empty, 1, b1 - b0 + 1)
    starts = jnp.concatenate([jnp.zeros(1, jnp.int32),
                              jnp.cumsum(cnt)[:-1].astype(jnp.int32)])

    n_pairs = nEb + nNb - 1
    p = jnp.arange(n_pairs, dtype=jnp.int32)
    j = jnp.clip(jnp.searchsorted(starts, p, side='right').astype(jnp.int32)
                 - 1, 0, nNb - 1)
    boff = p - starts[j]
    eb = jnp.clip(b0[j] + boff, 0, nEb - 1).astype(jnp.int32)
    va = ((~empty[j]) & (boff <= b1[j] - b0[j])).astype(jnp.int32)
    nb = j
    fi = (p == starts[j]).astype(jnp.int32)

    embP = (emb @ W_zij[:H]).astype(jnp.float32)
    embQ = (emb @ W_zij[H:]).astype(jnp.float32)
    embP = jnp.pad(embP, ((0, MAXZ_PAD - embP.shape[0]), (0, 0)))
    embQ = jnp.pad(embQ, ((0, MAXZ_PAD - embQ.shape[0]), (0, 0)))
    bz = b_zij.reshape(1, H)
    wias = jnp.concatenate([W_I, W_A, W_S], axis=1)
    bias = jnp.concatenate([b_I, b_A, b_S]).reshape(1, 3 * H)

    acc = _edge_scatter(eb, nb, fi, va, src3, az3, fr3,
                        embP, embQ, bz, wias, bias, npad,
                        interpret=interpret)
    acc = acc.reshape(npad, 10, H)

    perm = jnp.arange(3 * H).reshape(H, 3).T.reshape(-1)
    W_s2p = W_s2[:, perm]
    b_s2p = b_s2[perm]

    out9 = _node_phase(acc, W_t0, W_t1, W_t2, W_s1, b_s1, W_s2p, b_s2p,
                       ln_g, ln_b, interpret=interpret)
    out = jnp.transpose(out9[:n], (0, 2, 1)).reshape(n, H, 3, 3)
    return out


# speedup vs baseline: 1.3368x; 1.3314x over previous
import functools

import jax
import jax.numpy as jnp
import numpy as np
from jax.experimental import pallas as pl
from jax.experimental.pallas import tpu as pltpu

H = 128
R = 32
CUT = 0.5
CLO = 0.0
MAXZ_PAD = 128
BE = 2048
WN = 128


def _edge_scatter_block(eb_r, nb_r, fi_r, va_r, nw_r,
                        src_r, az_r, fr_r,
                        embP_r, embQ_r, bz_r, wias_r, bias_r,
                        out_r, m_r):
    p = pl.program_id(0)
    j = nb_r[p]
    first = fi_r[p]
    valid = va_r[p]

    be = src_r.shape[2]
    src = src_r[...].reshape(1, be)

    @pl.when(nw_r[p] == 1)
    def _():
        az = az_r[...].reshape(be, 2)
        azs = az[:, 0:1]
        azd = az[:, 1:2]
        ohS = (jax.lax.broadcasted_iota(jnp.int32, (be, MAXZ_PAD), 1) == azs
               ).astype(jnp.float32)
        ohD = (jax.lax.broadcasted_iota(jnp.int32, (be, MAXZ_PAD), 1) == azd
               ).astype(jnp.float32)
        zij = (jnp.dot(ohS, embP_r[...], preferred_element_type=jnp.float32)
               + jnp.dot(ohD, embQ_r[...], preferred_element_type=jnp.float32)
               + bz_r[...])

        fr = fr_r[...].reshape(be, 4)
        d = fr[:, 0:1]
        rcut = 0.5 * (jnp.cos(d * (np.pi / CUT)) + 1.0) * (d < CUT)
        alpha = 5.0 / (CUT - CLO)
        start = float(np.exp(-(CUT - CLO)))
        step = (1.0 - start) / (R - 1)
        means = (start + step *
                 jax.lax.broadcasted_iota(jnp.int32, (1, R), 1
                                          ).astype(jnp.float32))
        beta = (2.0 / R * (1.0 - start)) ** -2
        ex = jnp.exp(alpha * (CLO - d))
        rfv = jnp.exp(-beta * (ex - means) ** 2) * rcut

        pIAS = (jnp.dot(rfv, wias_r[...], preferred_element_type=jnp.float32)
                + bias_r[...])
        base = rcut * zij
        uI = pIAS[:, 0 * H:1 * H] * base
        uA = pIAS[:, 1 * H:2 * H] * base
        uS = pIAS[:, 2 * H:3 * H] * base

        dinv = 1.0 / d
        rxn = fr[:, 1:2] * dinv
        ryn = fr[:, 2:3] * dinv
        rzn = fr[:, 3:4] * dinv
        tr3 = (rxn * rxn + ryn * ryn + rzn * rzn) * (1.0 / 3.0)

        m_r[...] = jnp.concatenate([
            uI,
            rxn * uA, ryn * uA, rzn * uA,
            (rxn * rxn - tr3) * uS, (ryn * ryn - tr3) * uS,
            (rzn * rzn - tr3) * uS,
            (rxn * ryn) * uS, (rxn * rzn) * uS, (ryn * rzn) * uS,
        ], axis=1)

    rows = jax.lax.broadcasted_iota(jnp.int32, (WN, be), 0) + j * WN
    sel = (rows == src).astype(jnp.float32) * (valid == 1).astype(jnp.float32)
    contrib = jnp.dot(sel, m_r[...], preferred_element_type=jnp.float32)

    @pl.when(first == 1)
    def _():
        out_r[...] = jnp.zeros_like(out_r)

    out_r[...] += contrib


def _edge_scatter(eb, nb, fi, va, nw, src3, az3, fr3,
                  embP, embQ, bz, wias, bias, npad, interpret=False):
    n_pairs = eb.shape[0]
    be = src3.shape[2]
    full = lambda shp: pl.BlockSpec(shp, lambda p, e, n, f, v, w: (0, 0))
    grid_spec = pltpu.PrefetchScalarGridSpec(
        num_scalar_prefetch=5,
        grid=(n_pairs,),
        in_specs=[
            pl.BlockSpec((1, 1, be), lambda p, e, n, f, v, w: (e[p], 0, 0)),
            pl.BlockSpec((1, be, 2), lambda p, e, n, f, v, w: (e[p], 0, 0)),
            pl.BlockSpec((1, be, 4), lambda p, e, n, f, v, w: (e[p], 0, 0)),
            full((MAXZ_PAD, H)), full((MAXZ_PAD, H)), full((1, H)),
            full((R, 3 * H)), full((1, 3 * H)),
        ],
        out_specs=pl.BlockSpec((WN, 10 * H),
                               lambda p, e, n, f, v, w: (n[p], 0)),
        scratch_shapes=[pltpu.VMEM((be, 10 * H), jnp.float32)],
    )
    return pl.pallas_call(
        _edge_scatter_block,
        grid_spec=grid_spec,
        out_shape=jax.ShapeDtypeStruct((npad, 10 * H), jnp.float32),
        compiler_params=pltpu.CompilerParams(
            dimension_semantics=("arbitrary",)),
        interpret=interpret,
    )(eb, nb, fi, va, nw, src3, az3, fr3, embP, embQ, bz, wias, bias)


def _node_phase_block(acc_ref, wt0_ref, wt1_ref, wt2_ref, ws1_ref, bs1_ref,
                      ws2_ref, bs2_ref, g_ref, b_ref, out_ref):
    acc = acc_ref[...]
    sI = acc[:, 0, :]
    w0, w1, w2 = acc[:, 1, :], acc[:, 2, :], acc[:, 3, :]
    mxx, myy, mzz = acc[:, 4, :], acc[:, 5, :], acc[:, 6, :]
    mxy, mxz, myz = acc[:, 7, :], acc[:, 8, :], acc[:, 9, :]

    frob = (3.0 * sI * sI + 2.0 * (w0 * w0 + w1 * w1 + w2 * w2)
            + (mxx * mxx + myy * myy + mzz * mzz)
            + 2.0 * (mxy * mxy + mxz * mxz + myz * myz))

    mu = jnp.mean(frob, axis=-1, keepdims=True)
    var = jnp.mean((frob - mu) ** 2, axis=-1, keepdims=True)
    x = (frob - mu) * jax.lax.rsqrt(var + 1e-5) * g_ref[...] + b_ref[...]

    h1 = x @ ws1_ref[...] + bs1_ref[...]
    h1 = h1 * jax.nn.sigmoid(h1)
    h2 = h1 @ ws2_ref[...] + bs2_ref[...]
    h2 = h2 * jax.nn.sigmoid(h2)
    n0 = h2[:, 0 * H:1 * H]
    n1 = h2[:, 1 * H:2 * H]
    n2 = h2[:, 2 * H:3 * H]

    wt0 = wt0_ref[...]
    wt1 = wt1_ref[...]
    wt2 = wt2_ref[...]
    sIp = (sI @ wt0) * n0
    w0p = (w0 @ wt1) * n1
    w1p = (w1 @ wt1) * n1
    w2p = (w2 @ wt1) * n1
    mxxp = (mxx @ wt2) * n2
    myyp = (myy @ wt2) * n2
    mzzp = (mzz @ wt2) * n2
    mxyp = (mxy @ wt2) * n2
    mxzp = (mxz @ wt2) * n2
    myzp = (myz @ wt2) * n2

    out_ref[:, 0, :] = sIp + mxxp
    out_ref[:, 1, :] = -w2p + mxyp
    out_ref[:, 2, :] = w1p + mxzp
    out_ref[:, 3, :] = w2p + mxyp
    out_ref[:, 4, :] = sIp + myyp
    out_ref[:, 5, :] = -w0p + myzp
    out_ref[:, 6, :] = -w1p + mxzp
    out_ref[:, 7, :] = w0p + myzp
    out_ref[:, 8, :] = sIp + mzzp


def _node_phase(acc, W_t0, W_t1, W_t2, W_s1, b_s1, W_s2p, b_s2p, ln_g, ln_b,
                interpret=False):
    npad = acc.shape[0]
    bn = 64
    grid = (npad // bn,)
    full = lambda shp: pl.BlockSpec(shp, lambda i: (0,) * len(shp))
    return pl.pallas_call(
        _node_phase_block,
        grid=grid,
        in_specs=[
            pl.BlockSpec((bn, 10, H), lambda i: (i, 0, 0)),
            full((H, H)), full((H, H)), full((H, H)),
            full((H, 2 * H)), full((2 * H,)),
            full((2 * H, 3 * H)), full((3 * H,)),
            full((H,)), full((H,)),
        ],
        out_specs=pl.BlockSpec((bn, 9, H), lambda i: (i, 0, 0)),
        out_shape=jax.ShapeDtypeStruct((npad, 9, H), jnp.float32),
        interpret=interpret,
    )(acc, W_t0, W_t1, W_t2, W_s1, b_s1, W_s2p, b_s2p, ln_g, ln_b)


def kernel(atomic_numbers, pair_indices, d_ij, r_ij, emb, W_zij, b_zij,
           W_I, b_I, W_A, b_A, W_S, b_S, W_t0, W_t1, W_t2,
           W_s1, b_s1, W_s2, b_s2, ln_g, ln_b, *, interpret=False):
    n = atomic_numbers.shape[0]
    e = d_ij.shape[0]
    src = pair_indices[0]
    dst = pair_indices[1]

    eidx = jnp.arange(e, dtype=jnp.uint32)
    key = (src.astype(jnp.uint32) << 18) | eidx
    skey = jnp.sort(key)
    srcs = (skey >> 18).astype(jnp.int32)
    order = (skey & jnp.uint32(0x3FFFF)).astype(jnp.int32)
    azs = atomic_numbers[srcs].astype(jnp.int32)
    azd = atomic_numbers[dst[order]].astype(jnp.int32)
    frpre = jnp.stack([d_ij[:, 0], r_ij[:, 0], r_ij[:, 1], r_ij[:, 2]],
                      axis=-1)
    frs = frpre[order]
    ds = frs[:, 0]

    nEb = (e + BE - 1) // BE
    epad = nEb * BE
    npad = ((n + WN - 1) // WN) * WN
    nNb = npad // WN

    pe = epad - e
    srcp = jnp.pad(srcs.astype(jnp.int32), (0, pe), constant_values=n)
    azsp = jnp.pad(azs, (0, pe))
    azdp = jnp.pad(azd, (0, pe))
    frp = jnp.pad(frs, ((0, pe), (0, 0)))
    frp = frp.at[e:, 0].set(1.0) if pe else frp

    src3 = srcp.reshape(nEb, 1, BE)
    az3 = jnp.stack([azsp, azdp], axis=-1).reshape(nEb, BE, 2)
    fr3 = frp.reshape(nEb, BE, 4)

    wb = jnp.arange(nNb, dtype=jnp.int32)
    e0 = jnp.searchsorted(srcp, wb * WN).astype(jnp.int32)
    e1 = jnp.searchsorted(srcp, wb * WN + WN).astype(jnp.int32)
    empty = e1 == e0
    b0 = jnp.clip(e0 // BE, 0, nEb - 1)
    b1 = jnp.where(empty, b0, jnp.clip((e1 - 1) // BE, 0, nEb - 1))
    cnt = jnp.where(empty, 1, b1 - b0 + 1)
    starts = jnp.concatenate([jnp.zeros(1, jnp.int32),
                              jnp.cumsum(cnt)[:-1].astype(jnp.int32)])

    n_pairs = nEb + nNb - 1
    p = jnp.arange(n_pairs, dtype=jnp.int32)
    j = jnp.clip(jnp.searchsorted(starts, p, side='right').astype(jnp.int32)
                 - 1, 0, nNb - 1)
    boff = p - starts[j]
    eb = jnp.clip(b0[j] + boff, 0, nEb - 1).astype(jnp.int32)
    va = ((~empty[j]) & (boff <= b1[j] - b0[j])).astype(jnp.int32)
    nb = j
    fi = (p == starts[j]).astype(jnp.int32)
    nw = jnp.concatenate([jnp.ones(1, jnp.int32),
                          (eb[1:] != eb[:-1]).astype(jnp.int32)])

    embP = (emb @ W_zij[:H]).astype(jnp.float32)
    embQ = (emb @ W_zij[H:]).astype(jnp.float32)
    embP = jnp.pad(embP, ((0, MAXZ_PAD - embP.shape[0]), (0, 0)))
    embQ = jnp.pad(embQ, ((0, MAXZ_PAD - embQ.shape[0]), (0, 0)))
    bz = b_zij.reshape(1, H)
    wias = jnp.concatenate([W_I, W_A, W_S], axis=1)
    bias = jnp.concatenate([b_I, b_A, b_S]).reshape(1, 3 * H)

    acc = _edge_scatter(eb, nb, fi, va, nw, src3, az3, fr3,
                        embP, embQ, bz, wias, bias, npad,
                        interpret=interpret)
    acc = acc.reshape(npad, 10, H)

    perm = jnp.arange(3 * H).reshape(H, 3).T.reshape(-1)
    W_s2p = W_s2[:, perm]
    b_s2p = b_s2[perm]

    out9 = _node_phase(acc, W_t0, W_t1, W_t2, W_s1, b_s1, W_s2p, b_s2p,
                       ln_g, ln_b, interpret=interpret)
    out = jnp.transpose(out9[:n], (0, 2, 1)).reshape(n, H, 3, 3)
    return out


# speedup vs baseline: 1.3404x; 1.0027x over previous
import functools

import jax
import jax.numpy as jnp
import numpy as np
from jax.experimental import pallas as pl
from jax.experimental.pallas import tpu as pltpu

H = 128
R = 32
CUT = 0.5
CLO = 0.0
MAXZ_PAD = 128
BE = 2048
WN = 128


def _edge_scatter_block(eb_r, nb_r, fi_r, va_r, nw_r,
                        src_r, az_r, fr_r,
                        embP_r, embQ_r, bz_r, wias_r, bias_r,
                        out_r, m_r):
    p = pl.program_id(0)
    j = nb_r[p]
    first = fi_r[p]
    valid = va_r[p]

    be = src_r.shape[2]
    src = src_r[...].reshape(1, be)

    @pl.when(nw_r[p] == 1)
    def _():
        az = az_r[...].reshape(be, 2)
        azs = az[:, 0:1]
        azd = az[:, 1:2]
        ohS = (jax.lax.broadcasted_iota(jnp.int32, (be, MAXZ_PAD), 1) == azs
               ).astype(jnp.bfloat16)
        ohD = (jax.lax.broadcasted_iota(jnp.int32, (be, MAXZ_PAD), 1) == azd
               ).astype(jnp.bfloat16)
        zij = (jnp.dot(ohS, embP_r[...], preferred_element_type=jnp.float32)
               + jnp.dot(ohD, embQ_r[...], preferred_element_type=jnp.float32)
               + bz_r[...])

        fr = fr_r[...].reshape(be, 4)
        d = fr[:, 0:1]
        rcut = 0.5 * (jnp.cos(d * (np.pi / CUT)) + 1.0) * (d < CUT)
        alpha = 5.0 / (CUT - CLO)
        start = float(np.exp(-(CUT - CLO)))
        step = (1.0 - start) / (R - 1)
        means = (start + step *
                 jax.lax.broadcasted_iota(jnp.int32, (1, R), 1
                                          ).astype(jnp.float32))
        beta = (2.0 / R * (1.0 - start)) ** -2
        ex = jnp.exp(alpha * (CLO - d))
        rfv = jnp.exp(-beta * (ex - means) ** 2) * rcut

        pIAS = (jnp.dot(rfv.astype(jnp.bfloat16), wias_r[...],
                        preferred_element_type=jnp.float32)
                + bias_r[...])
        base = rcut * zij
        uI = pIAS[:, 0 * H:1 * H] * base
        uA = pIAS[:, 1 * H:2 * H] * base
        uS = pIAS[:, 2 * H:3 * H] * base

        dinv = 1.0 / d
        rxn = fr[:, 1:2] * dinv
        ryn = fr[:, 2:3] * dinv
        rzn = fr[:, 3:4] * dinv
        tr3 = (rxn * rxn + ryn * ryn + rzn * rzn) * (1.0 / 3.0)

        m_r[...] = jnp.concatenate([
            uI,
            rxn * uA, ryn * uA, rzn * uA,
            (rxn * rxn - tr3) * uS, (ryn * ryn - tr3) * uS,
            (rzn * rzn - tr3) * uS,
            (rxn * ryn) * uS, (rxn * rzn) * uS, (ryn * rzn) * uS,
        ], axis=1).astype(jnp.bfloat16)

    rows = jax.lax.broadcasted_iota(jnp.int32, (WN, be), 0) + j * WN
    sel = ((rows == src) & (valid == 1)).astype(jnp.bfloat16)
    contrib = jnp.dot(sel, m_r[...], preferred_element_type=jnp.float32)

    @pl.when(first == 1)
    def _():
        out_r[...] = jnp.zeros_like(out_r)

    out_r[...] += contrib


def _edge_scatter(eb, nb, fi, va, nw, src3, az3, fr3,
                  embP, embQ, bz, wias, bias, npad, interpret=False):
    n_pairs = eb.shape[0]
    be = src3.shape[2]
    full = lambda shp: pl.BlockSpec(shp, lambda p, e, n, f, v, w: (0, 0))
    grid_spec = pltpu.PrefetchScalarGridSpec(
        num_scalar_prefetch=5,
        grid=(n_pairs,),
        in_specs=[
            pl.BlockSpec((1, 1, be), lambda p, e, n, f, v, w: (e[p], 0, 0)),
            pl.BlockSpec((1, be, 2), lambda p, e, n, f, v, w: (e[p], 0, 0)),
            pl.BlockSpec((1, be, 4), lambda p, e, n, f, v, w: (e[p], 0, 0)),
            full((MAXZ_PAD, H)), full((MAXZ_PAD, H)), full((1, H)),
            full((R, 3 * H)), full((1, 3 * H)),
        ],
        out_specs=pl.BlockSpec((WN, 10 * H),
                               lambda p, e, n, f, v, w: (n[p], 0)),
        scratch_shapes=[pltpu.VMEM((be, 10 * H), jnp.bfloat16)],
    )
    return pl.pallas_call(
        _edge_scatter_block,
        grid_spec=grid_spec,
        out_shape=jax.ShapeDtypeStruct((npad, 10 * H), jnp.float32),
        compiler_params=pltpu.CompilerParams(
            dimension_semantics=("arbitrary",)),
        interpret=interpret,
    )(eb, nb, fi, va, nw, src3, az3, fr3, embP, embQ, bz, wias, bias)


def _node_phase_block(acc_ref, wt0_ref, wt1_ref, wt2_ref, ws1_ref, bs1_ref,
                      ws2_ref, bs2_ref, g_ref, b_ref, out_ref):
    acc = acc_ref[...]
    sI = acc[:, 0, :]
    w0, w1, w2 = acc[:, 1, :], acc[:, 2, :], acc[:, 3, :]
    mxx, myy, mzz = acc[:, 4, :], acc[:, 5, :], acc[:, 6, :]
    mxy, mxz, myz = acc[:, 7, :], acc[:, 8, :], acc[:, 9, :]

    frob = (3.0 * sI * sI + 2.0 * (w0 * w0 + w1 * w1 + w2 * w2)
            + (mxx * mxx + myy * myy + mzz * mzz)
            + 2.0 * (mxy * mxy + mxz * mxz + myz * myz))

    mu = jnp.mean(frob, axis=-1, keepdims=True)
    var = jnp.mean((frob - mu) ** 2, axis=-1, keepdims=True)
    x = (frob - mu) * jax.lax.rsqrt(var + 1e-5) * g_ref[...] + b_ref[...]

    h1 = x @ ws1_ref[...] + bs1_ref[...]
    h1 = h1 * jax.nn.sigmoid(h1)
    h2 = h1 @ ws2_ref[...] + bs2_ref[...]
    h2 = h2 * jax.nn.sigmoid(h2)
    n0 = h2[:, 0 * H:1 * H]
    n1 = h2[:, 1 * H:2 * H]
    n2 = h2[:, 2 * H:3 * H]

    wt0 = wt0_ref[...]
    wt1 = wt1_ref[...]
    wt2 = wt2_ref[...]
    sIp = (sI @ wt0) * n0
    w0p = (w0 @ wt1) * n1
    w1p = (w1 @ wt1) * n1
    w2p = (w2 @ wt1) * n1
    mxxp = (mxx @ wt2) * n2
    myyp = (myy @ wt2) * n2
    mzzp = (mzz @ wt2) * n2
    mxyp = (mxy @ wt2) * n2
    mxzp = (mxz @ wt2) * n2
    myzp = (myz @ wt2) * n2

    out_ref[:, 0, :] = sIp + mxxp
    out_ref[:, 1, :] = -w2p + mxyp
    out_ref[:, 2, :] = w1p + mxzp
    out_ref[:, 3, :] = w2p + mxyp
    out_ref[:, 4, :] = sIp + myyp
    out_ref[:, 5, :] = -w0p + myzp
    out_ref[:, 6, :] = -w1p + mxzp
    out_ref[:, 7, :] = w0p + myzp
    out_ref[:, 8, :] = sIp + mzzp


def _node_phase(acc, W_t0, W_t1, W_t2, W_s1, b_s1, W_s2p, b_s2p, ln_g, ln_b,
                interpret=False):
    npad = acc.shape[0]
    bn = 64
    grid = (npad // bn,)
    full = lambda shp: pl.BlockSpec(shp, lambda i: (0,) * len(shp))
    return pl.pallas_call(
        _node_phase_block,
        grid=grid,
        in_specs=[
            pl.BlockSpec((bn, 10, H), lambda i: (i, 0, 0)),
            full((H, H)), full((H, H)), full((H, H)),
            full((H, 2 * H)), full((2 * H,)),
            full((2 * H, 3 * H)), full((3 * H,)),
            full((H,)), full((H,)),
        ],
        out_specs=pl.BlockSpec((bn, 9, H), lambda i: (i, 0, 0)),
        out_shape=jax.ShapeDtypeStruct((npad, 9, H), jnp.float32),
        interpret=interpret,
    )(acc, W_t0, W_t1, W_t2, W_s1, b_s1, W_s2p, b_s2p, ln_g, ln_b)


def kernel(atomic_numbers, pair_indices, d_ij, r_ij, emb, W_zij, b_zij,
           W_I, b_I, W_A, b_A, W_S, b_S, W_t0, W_t1, W_t2,
           W_s1, b_s1, W_s2, b_s2, ln_g, ln_b, *, interpret=False):
    n = atomic_numbers.shape[0]
    e = d_ij.shape[0]
    src = pair_indices[0]
    dst = pair_indices[1]

    eidx = jnp.arange(e, dtype=jnp.uint32)
    key = (src.astype(jnp.uint32) << 18) | eidx
    skey = jnp.sort(key)
    srcs = (skey >> 18).astype(jnp.int32)
    order = (skey & jnp.uint32(0x3FFFF)).astype(jnp.int32)
    azs = atomic_numbers[srcs].astype(jnp.int32)
    azd = atomic_numbers[dst[order]].astype(jnp.int32)
    frpre = jnp.stack([d_ij[:, 0], r_ij[:, 0], r_ij[:, 1], r_ij[:, 2]],
                      axis=-1)
    frs = frpre[order]
    ds = frs[:, 0]

    nEb = (e + BE - 1) // BE
    epad = nEb * BE
    npad = ((n + WN - 1) // WN) * WN
    nNb = npad // WN

    pe = epad - e
    srcp = jnp.pad(srcs.astype(jnp.int32), (0, pe), constant_values=n)
    azsp = jnp.pad(azs, (0, pe))
    azdp = jnp.pad(azd, (0, pe))
    frp = jnp.pad(frs, ((0, pe), (0, 0)))
    frp = frp.at[e:, 0].set(1.0) if pe else frp

    src3 = srcp.reshape(nEb, 1, BE)
    az3 = jnp.stack([azsp, azdp], axis=-1).reshape(nEb, BE, 2)
    fr3 = frp.reshape(nEb, BE, 4)

    wb = jnp.arange(nNb, dtype=jnp.int32)
    e0 = jnp.searchsorted(srcp, wb * WN).astype(jnp.int32)
    e1 = jnp.searchsorted(srcp, wb * WN + WN).astype(jnp.int32)
    empty = e1 == e0
    b0 = jnp.clip(e0 // BE, 0, nEb - 1)
    b1 = jnp.where(empty, b0, jnp.clip((e1 - 1) // BE, 0, nEb - 1))
    cnt = jnp.where(empty, 1, b1 - b0 + 1)
    starts = jnp.concatenate([jnp.zeros(1, jnp.int32),
                              jnp.cumsum(cnt)[:-1].astype(jnp.int32)])

    n_pairs = nEb + nNb - 1
    p = jnp.arange(n_pairs, dtype=jnp.int32)
    j = jnp.clip(jnp.searchsorted(starts, p, side='right').astype(jnp.int32)
                 - 1, 0, nNb - 1)
    boff = p - starts[j]
    eb = jnp.clip(b0[j] + boff, 0, nEb - 1).astype(jnp.int32)
    va = ((~empty[j]) & (boff <= b1[j] - b0[j])).astype(jnp.int32)
    nb = j
    fi = (p == starts[j]).astype(jnp.int32)
    nw = jnp.concatenate([jnp.ones(1, jnp.int32),
                          (eb[1:] != eb[:-1]).astype(jnp.int32)])

    embP = (emb @ W_zij[:H]).astype(jnp.float32)
    embQ = (emb @ W_zij[H:]).astype(jnp.float32)
    embP = jnp.pad(embP, ((0, MAXZ_PAD - embP.shape[0]), (0, 0))
                   ).astype(jnp.bfloat16)
    embQ = jnp.pad(embQ, ((0, MAXZ_PAD - embQ.shape[0]), (0, 0))
                   ).astype(jnp.bfloat16)
    bz = b_zij.reshape(1, H)
    wias = jnp.concatenate([W_I, W_A, W_S], axis=1).astype(jnp.bfloat16)
    bias = jnp.concatenate([b_I, b_A, b_S]).reshape(1, 3 * H)

    acc = _edge_scatter(eb, nb, fi, va, nw, src3, az3, fr3,
                        embP, embQ, bz, wias, bias, npad,
                        interpret=interpret)
    acc = acc.reshape(npad, 10, H)

    perm = jnp.arange(3 * H).reshape(H, 3).T.reshape(-1)
    W_s2p = W_s2[:, perm]
    b_s2p = b_s2[perm]

    out9 = _node_phase(acc, W_t0, W_t1, W_t2, W_s1, b_s1, W_s2p, b_s2p,
                       ln_g, ln_b, interpret=interpret)
    out = jnp.transpose(out9[:n], (0, 2, 1)).reshape(n, H, 3, 3)
    return out
